# Initial kernel scaffold; baseline (speedup 1.0000x reference)
import functools

import jax
import jax.numpy as jnp
from jax import lax
from jax.experimental import pallas as pl
from jax.experimental.pallas import tpu as pltpu
from jax.experimental.pallas import tpu_sc as plsc

NUM_CORES = 2
NUM_SUBCORES = 16
NUM_WORKERS = NUM_CORES * NUM_SUBCORES
LANES = 16


def _make_kernel(B, L, V, D):
    rpw = B // NUM_WORKERS
    assert B % NUM_WORKERS == 0
    assert D == 2 * LANES
    assert L % 8 == 0 and L > 128 and L <= 256
    l_hi = L - 128
    n_full = L // LANES
    l_tail = L - n_full * LANES

    mesh = plsc.VectorSubcoreMesh(core_axis_name="c", subcore_axis_name="s")

    @functools.partial(
        pl.kernel,
        out_type=jax.ShapeDtypeStruct((B, D), jnp.float32),
        mesh=mesh,
        scratch_types=[
            pltpu.VMEM((rpw * L,), jnp.int32),
            pltpu.VMEM((L, D), jnp.float32),
            pltpu.VMEM((L, D), jnp.float32),
            pltpu.VMEM((rpw, D), jnp.float32),
            pltpu.SemaphoreType.DMA,
            pltpu.SemaphoreType.DMA,
        ],
    )
    def run(ids_hbm, table_hbm, out_hbm, ids_v, buf0, buf1, out_v, sem0, sem1):
        wid = lax.axis_index("s") * NUM_CORES + lax.axis_index("c")
        row0 = wid * rpw
        pltpu.sync_copy(ids_hbm.at[pl.ds(row0 * L, rpw * L)], ids_v)

        def issue(r, buf, sem):
            off = r * L
            pltpu.async_copy(
                table_hbm.at[ids_v.at[pl.ds(off, 128)]],
                buf.at[pl.ds(0, 128)], sem)
            pltpu.async_copy(
                table_hbm.at[ids_v.at[pl.ds(off + 128, l_hi)]],
                buf.at[pl.ds(128, l_hi)], sem)

        def wait_buf(buf, sem):
            pltpu.make_async_copy(table_hbm.at[pl.ds(0, L)], buf, sem).wait()

        lane = lax.iota(jnp.int32, LANES)

        def compute(r, buf):
            def sum_body(j, accs):
                a0, a1 = accs
                return (a0 + buf[j, pl.ds(0, LANES)],
                        a1 + buf[j, pl.ds(LANES, LANES)])
            a0, a1 = lax.fori_loop(
                0, L, sum_body,
                (jnp.zeros(LANES, jnp.float32), jnp.zeros(LANES, jnp.float32)),
                unroll=8)

            off = r * L

            def cnt_body(c, cnt):
                v = ids_v[pl.ds(off + c * LANES, LANES)]
                return cnt + plsc.all_reduce_population_count(v != 0)
            cnt = lax.fori_loop(0, n_full, cnt_body,
                                jnp.zeros(LANES, jnp.int32), unroll=4)
            if l_tail:
                tail = ids_v[pl.ds(off + L - LANES, LANES)]
                cnt = cnt + plsc.all_reduce_population_count(
                    (tail != 0) & (lane >= LANES - l_tail))

            inv = 1.0 / jnp.maximum(cnt.astype(jnp.float32), 1.0)
            out_v[r, pl.ds(0, LANES)] = a0 * inv
            out_v[r, pl.ds(LANES, LANES)] = a1 * inv

        issue(0, buf0, sem0)

        def outer(g, carry):
            r0 = g * 2
            issue(r0 + 1, buf1, sem1)
            wait_buf(buf0, sem0)
            compute(r0, buf0)

            @pl.when(r0 + 2 < rpw)
            def _():
                issue(r0 + 2, buf0, sem0)

            wait_buf(buf1, sem1)
            compute(r0 + 1, buf1)
            return carry

        lax.fori_loop(0, rpw // 2, outer, 0)
        pltpu.sync_copy(out_v, out_hbm.at[pl.ds(row0, rpw)])

    return run


def kernel(token_ids, table):
    B, L = token_ids.shape
    V, D = table.shape
    ids_flat = token_ids.reshape(-1).astype(jnp.int32)
    run = _make_kernel(B, L, V, D)
    return run(ids_flat, table)


# speedup vs baseline: 2.3109x; 2.3109x over previous
import functools

import jax
import jax.numpy as jnp
from jax import lax
from jax.experimental import pallas as pl
from jax.experimental.pallas import tpu as pltpu
from jax.experimental.pallas import tpu_sc as plsc

NUM_CORES = 2
NUM_SUBCORES = 16
NUM_WORKERS = NUM_CORES * NUM_SUBCORES
LANES = 16


def _make_kernel(B, L, V, D):
    rpw = B // NUM_WORKERS
    assert B % NUM_WORKERS == 0
    assert D == 2 * LANES
    assert L % 8 == 0 and L > 128 and L <= 256
    l_hi = L - 128
    n_full = L // LANES
    l_tail = L - n_full * LANES

    mesh = plsc.VectorSubcoreMesh(core_axis_name="c", subcore_axis_name="s")

    @functools.partial(
        pl.kernel,
        out_type=jax.ShapeDtypeStruct((B, D), jnp.float32),
        mesh=mesh,
        compiler_params=pltpu.CompilerParams(
            needs_layout_passes=False, use_tc_tiling_on_sc=False),
        scratch_types=[
            pltpu.VMEM((rpw * L,), jnp.int32),
            pltpu.VMEM((L, D), jnp.float32),
            pltpu.VMEM((L, D), jnp.float32),
            pltpu.VMEM((rpw, D), jnp.float32),
            pltpu.VMEM((rpw,), jnp.float32),
            pltpu.SemaphoreType.DMA,
            pltpu.SemaphoreType.DMA,
        ],
    )
    def run(ids_hbm, table_hbm, out_hbm, ids_v, buf0, buf1, out_v, inv_v,
            sem0, sem1):
        wid = lax.axis_index("s") * NUM_CORES + lax.axis_index("c")
        row0 = wid * rpw
        pltpu.sync_copy(ids_hbm.at[pl.ds(row0 * L, rpw * L)], ids_v)

        def issue(r, buf, sem):
            off = r * L
            pltpu.async_copy(
                table_hbm.at[ids_v.at[pl.ds(off, 128)]],
                buf.at[pl.ds(0, 128)], sem)
            pltpu.async_copy(
                table_hbm.at[ids_v.at[pl.ds(off + 128, l_hi)]],
                buf.at[pl.ds(128, l_hi)], sem)

        def wait_buf(buf, sem):
            pltpu.make_async_copy(table_hbm.at[pl.ds(0, L)], buf, sem).wait()

        lane = lax.iota(jnp.int32, LANES)
        one = jnp.ones(LANES, jnp.float32)
        zero = jnp.zeros(LANES, jnp.float32)

        def count_group(g, carry):
            rowoff = lane * L + g * (LANES * L)

            def cbody(j, cnt):
                v = plsc.load_gather(ids_v, [rowoff + j])
                return cnt + jnp.where(v != 0, one, zero)
            cnt = lax.fori_loop(0, L, cbody, zero, unroll=8)
            inv_v[pl.ds(g * LANES, LANES)] = 1.0 / jnp.maximum(cnt, 1.0)
            return carry

        def compute(r, buf):
            def sum_body(j, accs):
                a0, a1 = accs
                return (a0 + buf[j, pl.ds(0, LANES)],
                        a1 + buf[j, pl.ds(LANES, LANES)])
            a0, a1 = lax.fori_loop(
                0, L, sum_body,
                (jnp.zeros(LANES, jnp.float32), jnp.zeros(LANES, jnp.float32)),
                unroll=8)
            inv = plsc.load_gather(inv_v, [jnp.full((LANES,), r, jnp.int32)])
            out_v[r, pl.ds(0, LANES)] = a0 * inv
            out_v[r, pl.ds(LANES, LANES)] = a1 * inv

        issue(0, buf0, sem0)
        issue(1, buf1, sem1)
        lax.fori_loop(0, rpw // LANES, count_group, 0)

        def outer(g, carry):
            r0 = g * 2
            wait_buf(buf0, sem0)
            compute(r0, buf0)

            @pl.when(r0 + 2 < rpw)
            def _():
                issue(r0 + 2, buf0, sem0)

            wait_buf(buf1, sem1)
            compute(r0 + 1, buf1)

            @pl.when(r0 + 3 < rpw)
            def _():
                issue(r0 + 3, buf1, sem1)
            return carry

        lax.fori_loop(0, rpw // 2, outer, 0)
        pltpu.sync_copy(out_v, out_hbm.at[pl.ds(row0, rpw)])

    return run


def kernel(token_ids, table):
    B, L = token_ids.shape
    V, D = table.shape
    ids_flat = token_ids.reshape(-1).astype(jnp.int32)
    run = _make_kernel(B, L, V, D)
    return run(ids_flat, table)


# speedup vs baseline: 2.4554x; 1.0625x over previous
import functools

import jax
import jax.numpy as jnp
from jax import lax
from jax.experimental import pallas as pl
from jax.experimental.pallas import tpu as pltpu
from jax.experimental.pallas import tpu_sc as plsc

NUM_CORES = 2
NUM_SUBCORES = 16
NUM_WORKERS = NUM_CORES * NUM_SUBCORES
LANES = 16


def _make_kernel(B, L, V, D):
    rpw = B // NUM_WORKERS
    assert B % NUM_WORKERS == 0
    assert D == 2 * LANES
    assert L % 8 == 0 and L > 128 and L <= 256
    l_hi = L - 128
    n_full = L // LANES
    l_tail = L - n_full * LANES

    mesh = plsc.VectorSubcoreMesh(core_axis_name="c", subcore_axis_name="s")

    @functools.partial(
        pl.kernel,
        out_type=jax.ShapeDtypeStruct((B, D), jnp.float32),
        mesh=mesh,
        compiler_params=pltpu.CompilerParams(
            needs_layout_passes=False, use_tc_tiling_on_sc=False),
        scratch_types=[
            pltpu.VMEM((rpw * L,), jnp.int32),
            pltpu.VMEM((L, D), jnp.float32),
            pltpu.VMEM((L, D), jnp.float32),
            pltpu.VMEM((L, D), jnp.float32),
            pltpu.VMEM((L, D), jnp.float32),
            pltpu.VMEM((rpw, D), jnp.float32),
            pltpu.VMEM((rpw,), jnp.float32),
            pltpu.SemaphoreType.DMA,
            pltpu.SemaphoreType.DMA,
            pltpu.SemaphoreType.DMA,
            pltpu.SemaphoreType.DMA,
        ],
    )
    def run(ids_hbm, table_hbm, out_hbm, ids_v, buf0, buf1, buf2, buf3,
            out_v, inv_v, sem0, sem1, sem2, sem3):
        bufs = (buf0, buf1, buf2, buf3)
        sems = (sem0, sem1, sem2, sem3)
        nbuf = len(bufs)
        wid = lax.axis_index("s") * NUM_CORES + lax.axis_index("c")
        row0 = wid * rpw
        pltpu.sync_copy(ids_hbm.at[pl.ds(row0 * L, rpw * L)], ids_v)

        def issue(r, buf, sem):
            off = r * L
            pltpu.async_copy(
                table_hbm.at[ids_v.at[pl.ds(off, 128)]],
                buf.at[pl.ds(0, 128)], sem)
            pltpu.async_copy(
                table_hbm.at[ids_v.at[pl.ds(off + 128, l_hi)]],
                buf.at[pl.ds(128, l_hi)], sem)

        def wait_buf(buf, sem):
            pltpu.make_async_copy(table_hbm.at[pl.ds(0, L)], buf, sem).wait()

        lane = lax.iota(jnp.int32, LANES)
        one = jnp.ones(LANES, jnp.float32)
        zero = jnp.zeros(LANES, jnp.float32)

        def count_group(g, carry):
            rowoff = lane * L + g * (LANES * L)

            def cbody(j, cnt):
                v = plsc.load_gather(ids_v, [rowoff + j])
                return cnt + jnp.where(v != 0, one, zero)
            cnt = lax.fori_loop(0, L, cbody, zero, unroll=8)
            inv_v[pl.ds(g * LANES, LANES)] = 1.0 / jnp.maximum(cnt, 1.0)
            return carry

        def compute(r, buf):
            def sum_body(j, accs):
                a0, a1 = accs
                return (a0 + buf[j, pl.ds(0, LANES)],
                        a1 + buf[j, pl.ds(LANES, LANES)])
            a0, a1 = lax.fori_loop(
                0, L, sum_body,
                (jnp.zeros(LANES, jnp.float32), jnp.zeros(LANES, jnp.float32)),
                unroll=8)
            inv = plsc.load_gather(inv_v, [jnp.full((LANES,), r, jnp.int32)])
            out_v[r, pl.ds(0, LANES)] = a0 * inv
            out_v[r, pl.ds(LANES, LANES)] = a1 * inv

        for k in range(nbuf):
            issue(k, bufs[k], sems[k])
        lax.fori_loop(0, rpw // LANES, count_group, 0)

        def outer(g, carry):
            r0 = g * nbuf
            for k in range(nbuf):
                wait_buf(bufs[k], sems[k])
                compute(r0 + k, bufs[k])

                @pl.when(r0 + k + nbuf < rpw)
                def _():
                    issue(r0 + k + nbuf, bufs[k], sems[k])
            return carry

        lax.fori_loop(0, rpw // nbuf, outer, 0)
        pltpu.sync_copy(out_v, out_hbm.at[pl.ds(row0, rpw)])

    return run


def kernel(token_ids, table):
    B, L = token_ids.shape
    V, D = table.shape
    ids_flat = token_ids.reshape(-1).astype(jnp.int32)
    run = _make_kernel(B, L, V, D)
    return run(ids_flat, table)


# speedup vs baseline: 2.7947x; 1.1382x over previous
import functools

import jax
import jax.numpy as jnp
from jax import lax
from jax.experimental import pallas as pl
from jax.experimental.pallas import tpu as pltpu
from jax.experimental.pallas import tpu_sc as plsc

NUM_CORES = 2
NUM_SUBCORES = 16
NUM_WORKERS = NUM_CORES * NUM_SUBCORES
LANES = 16


PACK_BR = 2048


def _make_packer(V, D):
    BR = PACK_BR
    nblk = (V + BR - 1) // BR
    S = BR // 4

    def body(in_ref, out_ref):
        y = jnp.transpose(in_ref[...])
        v2 = jnp.concatenate(
            [y[b * S:(b + 1) * S, :] for b in range(4)], axis=1)
        out_ref[...] = v2.reshape(BR * D)

    return pl.pallas_call(
        body,
        grid=(nblk,),
        in_specs=[pl.BlockSpec((D, BR), lambda i: (0, i))],
        out_specs=pl.BlockSpec((BR * D,), lambda i: (i,)),
        out_shape=jax.ShapeDtypeStruct((nblk * BR * D,), jnp.float32),
    )


def _remap_ids(ids):
    h = ids >> 11
    q = ids & 2047
    b = q >> 9
    i = q & 511
    return (h << 11) + (i << 2) + b


def _make_kernel(B, L, V, D):
    rpw = B // NUM_WORKERS
    assert B % NUM_WORKERS == 0
    assert D == 2 * LANES
    assert L % 8 == 0 and L > 128 and L <= 256
    l_hi = L - 128
    n_full = L // LANES
    l_tail = L - n_full * LANES

    mesh = plsc.VectorSubcoreMesh(core_axis_name="c", subcore_axis_name="s")

    @functools.partial(
        pl.kernel,
        out_type=jax.ShapeDtypeStruct((B, D), jnp.float32),
        mesh=mesh,
        compiler_params=pltpu.CompilerParams(
            needs_layout_passes=False, use_tc_tiling_on_sc=False),
        scratch_types=[
            pltpu.VMEM((rpw * L,), jnp.int32),
            pltpu.VMEM((L, D), jnp.float32),
            pltpu.VMEM((L, D), jnp.float32),
            pltpu.VMEM((L, D), jnp.float32),
            pltpu.VMEM((L, D), jnp.float32),
            pltpu.VMEM((rpw, D), jnp.float32),
            pltpu.VMEM((rpw,), jnp.float32),
            pltpu.SemaphoreType.DMA,
            pltpu.SemaphoreType.DMA,
            pltpu.SemaphoreType.DMA,
            pltpu.SemaphoreType.DMA,
        ],
    )
    def run(ids_hbm, table_hbm, out_hbm, ids_v, buf0, buf1, buf2, buf3,
            out_v, inv_v, sem0, sem1, sem2, sem3):
        bufs = (buf0, buf1, buf2, buf3)
        sems = (sem0, sem1, sem2, sem3)
        nbuf = len(bufs)
        wid = lax.axis_index("s") * NUM_CORES + lax.axis_index("c")
        row0 = wid * rpw
        pltpu.sync_copy(ids_hbm.at[pl.ds(row0 * L, rpw * L)], ids_v)

        def issue(r, buf, sem):
            off = r * L
            pltpu.async_copy(
                table_hbm.at[ids_v.at[pl.ds(off, 128)]],
                buf.at[pl.ds(0, 128)], sem)
            pltpu.async_copy(
                table_hbm.at[ids_v.at[pl.ds(off + 128, l_hi)]],
                buf.at[pl.ds(128, l_hi)], sem)

        def wait_buf(buf, sem):
            pltpu.make_async_copy(table_hbm.at[pl.ds(0, L)], buf, sem).wait()

        lane = lax.iota(jnp.int32, LANES)
        one = jnp.ones(LANES, jnp.float32)
        zero = jnp.zeros(LANES, jnp.float32)

        def count_group(g, carry):
            rowoff = lane * L + g * (LANES * L)

            def cbody(j, cnt):
                v = plsc.load_gather(ids_v, [rowoff + j])
                return cnt + jnp.where(v != 0, one, zero)
            cnt = lax.fori_loop(0, L, cbody, zero, unroll=8)
            inv_v[pl.ds(g * LANES, LANES)] = 1.0 / jnp.maximum(cnt, 1.0)
            return carry

        def compute(r, buf):
            def sum_body(j, accs):
                a0, a1 = accs
                return (a0 + buf[j, pl.ds(0, LANES)],
                        a1 + buf[j, pl.ds(LANES, LANES)])
            a0, a1 = lax.fori_loop(
                0, L, sum_body,
                (jnp.zeros(LANES, jnp.float32), jnp.zeros(LANES, jnp.float32)),
                unroll=8)
            inv = plsc.load_gather(inv_v, [jnp.full((LANES,), r, jnp.int32)])
            out_v[r, pl.ds(0, LANES)] = a0 * inv
            out_v[r, pl.ds(LANES, LANES)] = a1 * inv

        for k in range(nbuf):
            issue(k, bufs[k], sems[k])
        lax.fori_loop(0, rpw // LANES, count_group, 0)

        def outer(g, carry):
            r0 = g * nbuf
            for k in range(nbuf):
                wait_buf(bufs[k], sems[k])
                compute(r0 + k, bufs[k])

                @pl.when(r0 + k + nbuf < rpw)
                def _():
                    issue(r0 + k + nbuf, bufs[k], sems[k])
            return carry

        lax.fori_loop(0, rpw // nbuf, outer, 0)
        pltpu.sync_copy(out_v, out_hbm.at[pl.ds(row0, rpw)])

    return run


def kernel(token_ids, table):
    B, L = token_ids.shape
    V, D = table.shape
    ids_flat = _remap_ids(token_ids.reshape(-1).astype(jnp.int32))
    packed = _make_packer(V, D)(table.T)
    vp = packed.shape[0] // D
    run = _make_kernel(B, L, vp, D)
    return run(ids_flat, packed.reshape(vp, D))


# speedup vs baseline: 3.4329x; 1.2283x over previous
import functools

import jax
import jax.numpy as jnp
from jax import lax
from jax.experimental import pallas as pl
from jax.experimental.pallas import tpu as pltpu
from jax.experimental.pallas import tpu_sc as plsc

NUM_CORES = 2
NUM_SUBCORES = 16
NUM_WORKERS = NUM_CORES * NUM_SUBCORES
LANES = 16


PACK_BR = 2048


def _make_packer(V, D):
    BR = PACK_BR
    nblk = (V + BR - 1) // BR
    S = BR // 4

    def body(in_ref, out_ref):
        x = in_ref[...]
        z = jnp.concatenate(
            [x[:, b * S:(b + 1) * S] for b in range(4)], axis=0)
        out_ref[...] = jnp.transpose(z).reshape(BR * D)

    return pl.pallas_call(
        body,
        grid=(nblk,),
        in_specs=[pl.BlockSpec((D, BR), lambda i: (0, i))],
        out_specs=pl.BlockSpec((BR * D,), lambda i: (i,)),
        out_shape=jax.ShapeDtypeStruct((nblk * BR * D,), jnp.float32),
    )


def _remap_ids(ids):
    h = ids >> 11
    q = ids & 2047
    b = q >> 9
    i = q & 511
    return (h << 11) + (i << 2) + b


def _make_kernel(B, L, V, D):
    rpw = B // NUM_WORKERS
    assert B % NUM_WORKERS == 0
    assert D == 2 * LANES
    assert L % 8 == 0 and L > 128 and L <= 256
    l_hi = L - 128
    n_full = L // LANES
    l_tail = L - n_full * LANES

    mesh = plsc.VectorSubcoreMesh(core_axis_name="c", subcore_axis_name="s")

    @functools.partial(
        pl.kernel,
        out_type=jax.ShapeDtypeStruct((B, D), jnp.float32),
        mesh=mesh,
        compiler_params=pltpu.CompilerParams(
            needs_layout_passes=False, use_tc_tiling_on_sc=False),
        scratch_types=[
            pltpu.VMEM((rpw * L,), jnp.int32),
            pltpu.VMEM((L, D), jnp.float32),
            pltpu.VMEM((L, D), jnp.float32),
            pltpu.VMEM((L, D), jnp.float32),
            pltpu.VMEM((L, D), jnp.float32),
            pltpu.VMEM((rpw, D), jnp.float32),
            pltpu.VMEM((rpw,), jnp.float32),
            pltpu.SemaphoreType.DMA,
            pltpu.SemaphoreType.DMA,
            pltpu.SemaphoreType.DMA,
            pltpu.SemaphoreType.DMA,
        ],
    )
    def run(ids_hbm, table_hbm, out_hbm, ids_v, buf0, buf1, buf2, buf3,
            out_v, inv_v, sem0, sem1, sem2, sem3):
        bufs = (buf0, buf1, buf2, buf3)
        sems = (sem0, sem1, sem2, sem3)
        nbuf = len(bufs)
        wid = lax.axis_index("s") * NUM_CORES + lax.axis_index("c")
        row0 = wid * rpw
        pltpu.sync_copy(ids_hbm.at[pl.ds(row0 * L, rpw * L)], ids_v)

        def issue(r, buf, sem):
            off = r * L
            pltpu.async_copy(
                table_hbm.at[ids_v.at[pl.ds(off, 128)]],
                buf.at[pl.ds(0, 128)], sem)
            pltpu.async_copy(
                table_hbm.at[ids_v.at[pl.ds(off + 128, l_hi)]],
                buf.at[pl.ds(128, l_hi)], sem)

        def wait_buf(buf, sem):
            pltpu.make_async_copy(table_hbm.at[pl.ds(0, L)], buf, sem).wait()

        lane = lax.iota(jnp.int32, LANES)
        one = jnp.ones(LANES, jnp.float32)
        zero = jnp.zeros(LANES, jnp.float32)

        def count_group(g, carry):
            rowoff = lane * L + g * (LANES * L)

            def cbody(j, cnt):
                v = plsc.load_gather(ids_v, [rowoff + j])
                return cnt + jnp.where(v != 0, one, zero)
            cnt = lax.fori_loop(0, L, cbody, zero, unroll=8)
            inv_v[pl.ds(g * LANES, LANES)] = 1.0 / jnp.maximum(cnt, 1.0)
            return carry

        def compute(r, buf):
            def sum_body(j, accs):
                a0, a1 = accs
                return (a0 + buf[j, pl.ds(0, LANES)],
                        a1 + buf[j, pl.ds(LANES, LANES)])
            a0, a1 = lax.fori_loop(
                0, L, sum_body,
                (jnp.zeros(LANES, jnp.float32), jnp.zeros(LANES, jnp.float32)),
                unroll=8)
            inv = plsc.load_gather(inv_v, [jnp.full((LANES,), r, jnp.int32)])
            out_v[r, pl.ds(0, LANES)] = a0 * inv
            out_v[r, pl.ds(LANES, LANES)] = a1 * inv

        for k in range(nbuf):
            issue(k, bufs[k], sems[k])
        lax.fori_loop(0, rpw // LANES, count_group, 0)

        def outer(g, carry):
            r0 = g * nbuf
            for k in range(nbuf):
                wait_buf(bufs[k], sems[k])
                compute(r0 + k, bufs[k])

                @pl.when(r0 + k + nbuf < rpw)
                def _():
                    issue(r0 + k + nbuf, bufs[k], sems[k])
            return carry

        lax.fori_loop(0, rpw // nbuf, outer, 0)
        pltpu.sync_copy(out_v, out_hbm.at[pl.ds(row0, rpw)])

    return run


def kernel(token_ids, table):
    B, L = token_ids.shape
    V, D = table.shape
    ids_flat = _remap_ids(token_ids.reshape(-1).astype(jnp.int32))
    packed = _make_packer(V, D)(table.T)
    vp = packed.shape[0] // D
    run = _make_kernel(B, L, vp, D)
    return run(ids_flat, packed.reshape(vp, D))


# speedup vs baseline: 6.3090x; 1.8378x over previous
import functools

import jax
import jax.numpy as jnp
from jax import lax
from jax.experimental import pallas as pl
from jax.experimental.pallas import tpu as pltpu
from jax.experimental.pallas import tpu_sc as plsc

NUM_CORES = 2
NUM_SUBCORES = 16
NUM_WORKERS = NUM_CORES * NUM_SUBCORES
LANES = 16


PACK_BR = 8192


def _make_packer(V, D):
    BR = PACK_BR
    nblk = (V + BR - 1) // BR
    CH = 2048
    S = CH // 4

    def body(in_ref, out_ref):
        for c in range(BR // CH):
            x = in_ref[:, pl.ds(c * CH, CH)]
            z = jnp.concatenate(
                [x[:, b * S:(b + 1) * S] for b in range(4)], axis=0)
            out_ref[pl.ds(c * CH * D, CH * D)] = (
                jnp.transpose(z).reshape(CH * D))

    return pl.pallas_call(
        body,
        grid=(nblk,),
        in_specs=[pl.BlockSpec((D, BR), lambda i: (0, i))],
        out_specs=pl.BlockSpec((BR * D,), lambda i: (i,)),
        out_shape=jax.ShapeDtypeStruct((nblk * BR * D,), jnp.float32),
    )


def _remap_ids(ids):
    h = ids >> 11
    q = ids & 2047
    b = q >> 9
    i = q & 511
    return (h << 11) + (i << 2) + b


def _make_kernel(B, L, V, D):
    rpw = B // NUM_WORKERS
    assert B % NUM_WORKERS == 0
    assert D == 2 * LANES
    assert L % 8 == 0 and L > 128 and L <= 256
    l_hi = L - 128
    n_full = L // LANES
    l_tail = L - n_full * LANES

    mesh = plsc.VectorSubcoreMesh(core_axis_name="c", subcore_axis_name="s")

    @functools.partial(
        pl.kernel,
        out_type=jax.ShapeDtypeStruct((B, D), jnp.float32),
        mesh=mesh,
        compiler_params=pltpu.CompilerParams(
            needs_layout_passes=False, use_tc_tiling_on_sc=False),
        scratch_types=[
            pltpu.VMEM((rpw * L,), jnp.int32),
            pltpu.VMEM((L, D), jnp.float32),
            pltpu.VMEM((L, D), jnp.float32),
            pltpu.VMEM((L, D), jnp.float32),
            pltpu.VMEM((L, D), jnp.float32),
            pltpu.VMEM((rpw, D), jnp.float32),
            pltpu.VMEM((rpw,), jnp.float32),
            pltpu.SemaphoreType.DMA,
            pltpu.SemaphoreType.DMA,
            pltpu.SemaphoreType.DMA,
            pltpu.SemaphoreType.DMA,
        ],
    )
    def run(ids_hbm, table_hbm, out_hbm, ids_v, buf0, buf1, buf2, buf3,
            out_v, inv_v, sem0, sem1, sem2, sem3):
        bufs = (buf0, buf1, buf2, buf3)
        sems = (sem0, sem1, sem2, sem3)
        nbuf = len(bufs)
        wid = lax.axis_index("s") * NUM_CORES + lax.axis_index("c")
        row0 = wid * rpw
        pltpu.sync_copy(ids_hbm.at[pl.ds(row0 * L, rpw * L)], ids_v)

        def issue(r, buf, sem):
            off = r * L
            pltpu.async_copy(
                table_hbm.at[ids_v.at[pl.ds(off, 128)]],
                buf.at[pl.ds(0, 128)], sem)
            pltpu.async_copy(
                table_hbm.at[ids_v.at[pl.ds(off + 128, l_hi)]],
                buf.at[pl.ds(128, l_hi)], sem)

        def wait_buf(buf, sem):
            pltpu.make_async_copy(table_hbm.at[pl.ds(0, L)], buf, sem).wait()

        lane = lax.iota(jnp.int32, LANES)
        one = jnp.ones(LANES, jnp.float32)
        zero = jnp.zeros(LANES, jnp.float32)

        def count_group(g, carry):
            rowoff = lane * L + g * (LANES * L)

            def cbody(j, cnt):
                v = plsc.load_gather(ids_v, [rowoff + j])
                return cnt + jnp.where(v != 0, one, zero)
            cnt = lax.fori_loop(0, L, cbody, zero, unroll=8)
            inv_v[pl.ds(g * LANES, LANES)] = 1.0 / jnp.maximum(cnt, 1.0)
            return carry

        def compute(r, buf):
            def sum_body(j, accs):
                a0, a1 = accs
                return (a0 + buf[j, pl.ds(0, LANES)],
                        a1 + buf[j, pl.ds(LANES, LANES)])
            a0, a1 = lax.fori_loop(
                0, L, sum_body,
                (jnp.zeros(LANES, jnp.float32), jnp.zeros(LANES, jnp.float32)),
                unroll=8)
            inv = plsc.load_gather(inv_v, [jnp.full((LANES,), r, jnp.int32)])
            out_v[r, pl.ds(0, LANES)] = a0 * inv
            out_v[r, pl.ds(LANES, LANES)] = a1 * inv

        for k in range(nbuf):
            issue(k, bufs[k], sems[k])
        lax.fori_loop(0, rpw // LANES, count_group, 0)

        def outer(g, carry):
            r0 = g * nbuf
            for k in range(nbuf):
                wait_buf(bufs[k], sems[k])
                compute(r0 + k, bufs[k])

                @pl.when(r0 + k + nbuf < rpw)
                def _():
                    issue(r0 + k + nbuf, bufs[k], sems[k])
            return carry

        lax.fori_loop(0, rpw // nbuf, outer, 0)
        pltpu.sync_copy(out_v, out_hbm.at[pl.ds(row0, rpw)])

    return run


def kernel(token_ids, table):
    B, L = token_ids.shape
    V, D = table.shape
    ids_flat = _remap_ids(token_ids.reshape(-1).astype(jnp.int32))
    packed = _make_packer(V, D)(table.T)
    vp = packed.shape[0] // D
    run = _make_kernel(B, L, vp, D)
    return run(ids_flat, packed.reshape(vp, D))


# speedup vs baseline: 7.6459x; 1.2119x over previous
import functools

import jax
import jax.numpy as jnp
from jax import lax
from jax.experimental import pallas as pl
from jax.experimental.pallas import tpu as pltpu
from jax.experimental.pallas import tpu_sc as plsc

NUM_CORES = 2
NUM_SUBCORES = 16
NUM_WORKERS = NUM_CORES * NUM_SUBCORES
LANES = 16


PACK_BR = 16384


def _make_packer(V, D):
    BR = PACK_BR
    nblk = (V + BR - 1) // BR
    CH = 2048
    S = CH // 4

    def body(in_ref, out_ref):
        for c in range(BR // CH):
            x = in_ref[:, pl.ds(c * CH, CH)]
            z = jnp.concatenate(
                [x[:, b * S:(b + 1) * S] for b in range(4)], axis=0)
            out_ref[pl.ds(c * CH * D, CH * D)] = (
                jnp.transpose(z).reshape(CH * D))

    return pl.pallas_call(
        body,
        grid=(nblk,),
        in_specs=[pl.BlockSpec((D, BR), lambda i: (0, i))],
        out_specs=pl.BlockSpec((BR * D,), lambda i: (i,)),
        out_shape=jax.ShapeDtypeStruct((nblk * BR * D,), jnp.float32),
    )


def _remap_ids(ids):
    h = ids >> 11
    q = ids & 2047
    b = q >> 9
    i = q & 511
    return (h << 11) + (i << 2) + b


def _make_kernel(B, L, V, D):
    rpw = B // NUM_WORKERS
    assert B % NUM_WORKERS == 0
    assert D == 2 * LANES
    assert L % 8 == 0 and L > 128 and L <= 256
    l_hi = L - 128
    n_full = L // LANES
    l_tail = L - n_full * LANES

    mesh = plsc.VectorSubcoreMesh(core_axis_name="c", subcore_axis_name="s")

    @functools.partial(
        pl.kernel,
        out_type=jax.ShapeDtypeStruct((B, D), jnp.float32),
        mesh=mesh,
        compiler_params=pltpu.CompilerParams(
            needs_layout_passes=False, use_tc_tiling_on_sc=False),
        scratch_types=[
            pltpu.VMEM((rpw * L,), jnp.int32),
            pltpu.VMEM((L, D), jnp.float32),
            pltpu.VMEM((L, D), jnp.float32),
            pltpu.VMEM((L, D), jnp.float32),
            pltpu.VMEM((L, D), jnp.float32),
            pltpu.VMEM((rpw, D), jnp.float32),
            pltpu.VMEM((rpw,), jnp.float32),
            pltpu.SemaphoreType.DMA,
            pltpu.SemaphoreType.DMA,
            pltpu.SemaphoreType.DMA,
            pltpu.SemaphoreType.DMA,
        ],
    )
    def run(ids_hbm, table_hbm, out_hbm, ids_v, buf0, buf1, buf2, buf3,
            out_v, inv_v, sem0, sem1, sem2, sem3):
        bufs = (buf0, buf1, buf2, buf3)
        sems = (sem0, sem1, sem2, sem3)
        nbuf = len(bufs)
        wid = lax.axis_index("s") * NUM_CORES + lax.axis_index("c")
        row0 = wid * rpw
        pltpu.sync_copy(ids_hbm.at[pl.ds(row0 * L, rpw * L)], ids_v)

        def issue(r, buf, sem):
            off = r * L
            pltpu.async_copy(
                table_hbm.at[ids_v.at[pl.ds(off, 128)]],
                buf.at[pl.ds(0, 128)], sem)
            pltpu.async_copy(
                table_hbm.at[ids_v.at[pl.ds(off + 128, l_hi)]],
                buf.at[pl.ds(128, l_hi)], sem)

        def wait_buf(buf, sem):
            pltpu.make_async_copy(table_hbm.at[pl.ds(0, L)], buf, sem).wait()

        lane = lax.iota(jnp.int32, LANES)
        one = jnp.ones(LANES, jnp.float32)
        zero = jnp.zeros(LANES, jnp.float32)

        def count_group(g, carry):
            rowoff = lane * L + g * (LANES * L)

            def cbody(j, cnt):
                v = plsc.load_gather(ids_v, [rowoff + j])
                return cnt + jnp.where(v != 0, one, zero)
            cnt = lax.fori_loop(0, L, cbody, zero, unroll=8)
            inv_v[pl.ds(g * LANES, LANES)] = 1.0 / jnp.maximum(cnt, 1.0)
            return carry

        def compute(r, buf):
            def sum_body(j, accs):
                a0, a1 = accs
                return (a0 + buf[j, pl.ds(0, LANES)],
                        a1 + buf[j, pl.ds(LANES, LANES)])
            a0, a1 = lax.fori_loop(
                0, L, sum_body,
                (jnp.zeros(LANES, jnp.float32), jnp.zeros(LANES, jnp.float32)),
                unroll=8)
            inv = plsc.load_gather(inv_v, [jnp.full((LANES,), r, jnp.int32)])
            out_v[r, pl.ds(0, LANES)] = a0 * inv
            out_v[r, pl.ds(LANES, LANES)] = a1 * inv

        for k in range(nbuf):
            issue(k, bufs[k], sems[k])
        lax.fori_loop(0, rpw // LANES, count_group, 0)

        def outer(g, carry):
            r0 = g * nbuf
            for k in range(nbuf):
                wait_buf(bufs[k], sems[k])
                compute(r0 + k, bufs[k])

                @pl.when(r0 + k + nbuf < rpw)
                def _():
                    issue(r0 + k + nbuf, bufs[k], sems[k])
            return carry

        lax.fori_loop(0, rpw // nbuf, outer, 0)
        pltpu.sync_copy(out_v, out_hbm.at[pl.ds(row0, rpw)])

    return run


def kernel(token_ids, table):
    B, L = token_ids.shape
    V, D = table.shape
    ids_flat = _remap_ids(token_ids.reshape(-1).astype(jnp.int32))
    packed = _make_packer(V, D)(table.T)
    vp = packed.shape[0] // D
    run = _make_kernel(B, L, vp, D)
    return run(ids_flat, packed.reshape(vp, D))


# speedup vs baseline: 8.2986x; 1.0854x over previous
import functools

import jax
import jax.numpy as jnp
from jax import lax
from jax.experimental import pallas as pl
from jax.experimental.pallas import tpu as pltpu
from jax.experimental.pallas import tpu_sc as plsc

NUM_CORES = 2
NUM_SUBCORES = 16
NUM_WORKERS = NUM_CORES * NUM_SUBCORES
LANES = 16


PACK_BR = 32768


def _make_packer(V, D):
    BR = PACK_BR
    nblk = (V + BR - 1) // BR
    CH = 2048
    S = CH // 4

    def body(in_ref, out_ref):
        for c in range(BR // CH):
            x = in_ref[:, pl.ds(c * CH, CH)]
            z = jnp.concatenate(
                [x[:, b * S:(b + 1) * S] for b in range(4)], axis=0)
            out_ref[pl.ds(c * CH * D, CH * D)] = (
                jnp.transpose(z).reshape(CH * D))

    return pl.pallas_call(
        body,
        grid=(nblk,),
        in_specs=[pl.BlockSpec((D, BR), lambda i: (0, i))],
        out_specs=pl.BlockSpec((BR * D,), lambda i: (i,)),
        out_shape=jax.ShapeDtypeStruct((nblk * BR * D,), jnp.float32),
    )


def _remap_ids(ids):
    h = ids >> 11
    q = ids & 2047
    b = q >> 9
    i = q & 511
    return (h << 11) + (i << 2) + b


def _make_kernel(B, L, V, D):
    rpw = B // NUM_WORKERS
    assert B % NUM_WORKERS == 0
    assert D == 2 * LANES
    assert L % 8 == 0 and L > 128 and L <= 256
    l_hi = L - 128
    n_full = L // LANES
    l_tail = L - n_full * LANES

    mesh = plsc.VectorSubcoreMesh(core_axis_name="c", subcore_axis_name="s")

    @functools.partial(
        pl.kernel,
        out_type=jax.ShapeDtypeStruct((B, D), jnp.float32),
        mesh=mesh,
        compiler_params=pltpu.CompilerParams(
            needs_layout_passes=False, use_tc_tiling_on_sc=False),
        scratch_types=[
            pltpu.VMEM((rpw * L,), jnp.int32),
            pltpu.VMEM((L, D), jnp.float32),
            pltpu.VMEM((L, D), jnp.float32),
            pltpu.VMEM((L, D), jnp.float32),
            pltpu.VMEM((L, D), jnp.float32),
            pltpu.VMEM((rpw, D), jnp.float32),
            pltpu.VMEM((rpw,), jnp.float32),
            pltpu.SemaphoreType.DMA,
            pltpu.SemaphoreType.DMA,
            pltpu.SemaphoreType.DMA,
            pltpu.SemaphoreType.DMA,
        ],
    )
    def run(ids_hbm, table_hbm, out_hbm, ids_v, buf0, buf1, buf2, buf3,
            out_v, inv_v, sem0, sem1, sem2, sem3):
        bufs = (buf0, buf1, buf2, buf3)
        sems = (sem0, sem1, sem2, sem3)
        nbuf = len(bufs)
        wid = lax.axis_index("s") * NUM_CORES + lax.axis_index("c")
        row0 = wid * rpw
        pltpu.sync_copy(ids_hbm.at[pl.ds(row0 * L, rpw * L)], ids_v)

        def issue(r, buf, sem):
            off = r * L
            pltpu.async_copy(
                table_hbm.at[ids_v.at[pl.ds(off, 128)]],
                buf.at[pl.ds(0, 128)], sem)
            pltpu.async_copy(
                table_hbm.at[ids_v.at[pl.ds(off + 128, l_hi)]],
                buf.at[pl.ds(128, l_hi)], sem)

        def wait_buf(buf, sem):
            pltpu.make_async_copy(table_hbm.at[pl.ds(0, L)], buf, sem).wait()

        lane = lax.iota(jnp.int32, LANES)
        one = jnp.ones(LANES, jnp.float32)
        zero = jnp.zeros(LANES, jnp.float32)

        def count_group(g, carry):
            rowoff = lane * L + g * (LANES * L)

            def cbody(j, cnt):
                v = plsc.load_gather(ids_v, [rowoff + j])
                return cnt + jnp.where(v != 0, one, zero)
            cnt = lax.fori_loop(0, L, cbody, zero, unroll=8)
            inv_v[pl.ds(g * LANES, LANES)] = 1.0 / jnp.maximum(cnt, 1.0)
            return carry

        def compute(r, buf):
            def sum_body(j, accs):
                a0, a1 = accs
                return (a0 + buf[j, pl.ds(0, LANES)],
                        a1 + buf[j, pl.ds(LANES, LANES)])
            a0, a1 = lax.fori_loop(
                0, L, sum_body,
                (jnp.zeros(LANES, jnp.float32), jnp.zeros(LANES, jnp.float32)),
                unroll=8)
            inv = plsc.load_gather(inv_v, [jnp.full((LANES,), r, jnp.int32)])
            out_v[r, pl.ds(0, LANES)] = a0 * inv
            out_v[r, pl.ds(LANES, LANES)] = a1 * inv

        for k in range(nbuf):
            issue(k, bufs[k], sems[k])
        lax.fori_loop(0, rpw // LANES, count_group, 0)

        def outer(g, carry):
            r0 = g * nbuf
            for k in range(nbuf):
                wait_buf(bufs[k], sems[k])
                compute(r0 + k, bufs[k])

                @pl.when(r0 + k + nbuf < rpw)
                def _():
                    issue(r0 + k + nbuf, bufs[k], sems[k])
            return carry

        lax.fori_loop(0, rpw // nbuf, outer, 0)
        pltpu.sync_copy(out_v, out_hbm.at[pl.ds(row0, rpw)])

    return run


def kernel(token_ids, table):
    B, L = token_ids.shape
    V, D = table.shape
    ids_flat = _remap_ids(token_ids.reshape(-1).astype(jnp.int32))
    packed = _make_packer(V, D)(table.T)
    vp = packed.shape[0] // D
    run = _make_kernel(B, L, vp, D)
    return run(ids_flat, packed.reshape(vp, D))


# speedup vs baseline: 8.3647x; 1.0080x over previous
import functools

import jax
import jax.numpy as jnp
from jax import lax
from jax.experimental import pallas as pl
from jax.experimental.pallas import tpu as pltpu
from jax.experimental.pallas import tpu_sc as plsc

NUM_CORES = 2
NUM_SUBCORES = 16
NUM_WORKERS = NUM_CORES * NUM_SUBCORES
LANES = 16


PACK_BR = 65536


def _make_packer(V, D):
    BR = PACK_BR
    nblk = (V + BR - 1) // BR
    CH = 2048
    S = CH // 4

    def body(in_ref, out_ref):
        for c in range(BR // CH):
            x = in_ref[:, pl.ds(c * CH, CH)]
            z = jnp.concatenate(
                [x[:, b * S:(b + 1) * S] for b in range(4)], axis=0)
            out_ref[pl.ds(c * CH * D, CH * D)] = (
                jnp.transpose(z).reshape(CH * D))

    return pl.pallas_call(
        body,
        grid=(nblk,),
        in_specs=[pl.BlockSpec((D, BR), lambda i: (0, i))],
        out_specs=pl.BlockSpec((BR * D,), lambda i: (i,)),
        out_shape=jax.ShapeDtypeStruct((nblk * BR * D,), jnp.float32),
    )


def _remap_ids(ids):
    h = ids >> 11
    q = ids & 2047
    b = q >> 9
    i = q & 511
    return (h << 11) + (i << 2) + b


def _make_kernel(B, L, V, D):
    rpw = B // NUM_WORKERS
    assert B % NUM_WORKERS == 0
    assert D == 2 * LANES
    assert L % 8 == 0 and L > 128 and L <= 256
    l_hi = L - 128
    n_full = L // LANES
    l_tail = L - n_full * LANES

    mesh = plsc.VectorSubcoreMesh(core_axis_name="c", subcore_axis_name="s")

    @functools.partial(
        pl.kernel,
        out_type=jax.ShapeDtypeStruct((B, D), jnp.float32),
        mesh=mesh,
        compiler_params=pltpu.CompilerParams(
            needs_layout_passes=False, use_tc_tiling_on_sc=False),
        scratch_types=[
            pltpu.VMEM((rpw * L,), jnp.int32),
            pltpu.VMEM((L, D), jnp.float32),
            pltpu.VMEM((L, D), jnp.float32),
            pltpu.VMEM((L, D), jnp.float32),
            pltpu.VMEM((L, D), jnp.float32),
            pltpu.VMEM((rpw, D), jnp.float32),
            pltpu.VMEM((rpw,), jnp.float32),
            pltpu.SemaphoreType.DMA,
            pltpu.SemaphoreType.DMA,
            pltpu.SemaphoreType.DMA,
            pltpu.SemaphoreType.DMA,
        ],
    )
    def run(ids_hbm, table_hbm, out_hbm, ids_v, buf0, buf1, buf2, buf3,
            out_v, inv_v, sem0, sem1, sem2, sem3):
        bufs = (buf0, buf1, buf2, buf3)
        sems = (sem0, sem1, sem2, sem3)
        nbuf = len(bufs)
        wid = lax.axis_index("s") * NUM_CORES + lax.axis_index("c")
        row0 = wid * rpw
        pltpu.sync_copy(ids_hbm.at[pl.ds(row0 * L, rpw * L)], ids_v)

        def issue(r, buf, sem):
            off = r * L
            pltpu.async_copy(
                table_hbm.at[ids_v.at[pl.ds(off, 128)]],
                buf.at[pl.ds(0, 128)], sem)
            pltpu.async_copy(
                table_hbm.at[ids_v.at[pl.ds(off + 128, l_hi)]],
                buf.at[pl.ds(128, l_hi)], sem)

        def wait_buf(buf, sem):
            pltpu.make_async_copy(table_hbm.at[pl.ds(0, L)], buf, sem).wait()

        lane = lax.iota(jnp.int32, LANES)
        one = jnp.ones(LANES, jnp.float32)
        zero = jnp.zeros(LANES, jnp.float32)

        def count_group(g, carry):
            rowoff = lane * L + g * (LANES * L)

            def cbody(j, cnt):
                v = plsc.load_gather(ids_v, [rowoff + j])
                return cnt + jnp.where(v != 0, one, zero)
            cnt = lax.fori_loop(0, L, cbody, zero, unroll=8)
            inv_v[pl.ds(g * LANES, LANES)] = 1.0 / jnp.maximum(cnt, 1.0)
            return carry

        def compute(r, buf):
            def sum_body(j, accs):
                a0, a1 = accs
                return (a0 + buf[j, pl.ds(0, LANES)],
                        a1 + buf[j, pl.ds(LANES, LANES)])
            a0, a1 = lax.fori_loop(
                0, L, sum_body,
                (jnp.zeros(LANES, jnp.float32), jnp.zeros(LANES, jnp.float32)),
                unroll=8)
            inv = plsc.load_gather(inv_v, [jnp.full((LANES,), r, jnp.int32)])
            out_v[r, pl.ds(0, LANES)] = a0 * inv
            out_v[r, pl.ds(LANES, LANES)] = a1 * inv

        for k in range(nbuf):
            issue(k, bufs[k], sems[k])
        lax.fori_loop(0, rpw // LANES, count_group, 0)

        def outer(g, carry):
            r0 = g * nbuf
            for k in range(nbuf):
                wait_buf(bufs[k], sems[k])
                compute(r0 + k, bufs[k])

                @pl.when(r0 + k + nbuf < rpw)
                def _():
                    issue(r0 + k + nbuf, bufs[k], sems[k])
            return carry

        lax.fori_loop(0, rpw // nbuf, outer, 0)
        pltpu.sync_copy(out_v, out_hbm.at[pl.ds(row0, rpw)])

    return run


def kernel(token_ids, table):
    B, L = token_ids.shape
    V, D = table.shape
    ids_flat = _remap_ids(token_ids.reshape(-1).astype(jnp.int32))
    packed = _make_packer(V, D)(table.T)
    vp = packed.shape[0] // D
    run = _make_kernel(B, L, vp, D)
    return run(ids_flat, packed.reshape(vp, D))


# speedup vs baseline: 8.3703x; 1.0007x over previous
import functools

import jax
import jax.numpy as jnp
from jax import lax
from jax.experimental import pallas as pl
from jax.experimental.pallas import tpu as pltpu
from jax.experimental.pallas import tpu_sc as plsc

NUM_CORES = 2
NUM_SUBCORES = 16
NUM_WORKERS = NUM_CORES * NUM_SUBCORES
LANES = 16


PACK_BR = 65536


def _make_packer(V, D):
    BR = PACK_BR
    nblk = (V + BR - 1) // BR
    CH = 2048
    S = CH // 4

    def body(in_ref, out_ref):
        for c in range(BR // CH):
            x = in_ref[:, pl.ds(c * CH, CH)]
            z = jnp.concatenate(
                [x[:, b * S:(b + 1) * S] for b in range(4)], axis=0)
            out_ref[pl.ds(c * CH * D, CH * D)] = (
                jnp.transpose(z).reshape(CH * D))

    return pl.pallas_call(
        body,
        grid=(nblk,),
        in_specs=[pl.BlockSpec((D, BR), lambda i: (0, i))],
        out_specs=pl.BlockSpec((BR * D,), lambda i: (i,)),
        out_shape=jax.ShapeDtypeStruct((nblk * BR * D,), jnp.float32),
    )


def _remap_ids(ids):
    h = ids >> 11
    q = ids & 2047
    b = q >> 9
    i = q & 511
    return (h << 11) + (i << 2) + b


def _make_ids_packer(B, L, LP):
    BB = 512

    def body(in_ref, out_ref):
        p = _remap_ids(in_ref[...])
        z = jnp.concatenate(
            [p, jnp.zeros((LP - L, BB), jnp.int32)], axis=0)
        out_ref[...] = jnp.transpose(z).reshape(BB * LP)

    return pl.pallas_call(
        body,
        grid=(B // BB,),
        in_specs=[pl.BlockSpec((L, BB), lambda i: (0, i))],
        out_specs=pl.BlockSpec((BB * LP,), lambda i: (i,)),
        out_shape=jax.ShapeDtypeStruct((B * LP,), jnp.int32),
    )


def _make_kernel(B, L, LP, V, D):
    rpw = B // NUM_WORKERS
    assert B % NUM_WORKERS == 0
    assert D == 2 * LANES
    assert L % 8 == 0 and L > 128 and L <= 256
    l_hi = L - 128
    n_full = L // LANES
    l_tail = L - n_full * LANES

    mesh = plsc.VectorSubcoreMesh(core_axis_name="c", subcore_axis_name="s")

    @functools.partial(
        pl.kernel,
        out_type=jax.ShapeDtypeStruct((B, D), jnp.float32),
        mesh=mesh,
        compiler_params=pltpu.CompilerParams(
            needs_layout_passes=False, use_tc_tiling_on_sc=False),
        scratch_types=[
            pltpu.VMEM((rpw * LP,), jnp.int32),
            pltpu.VMEM((L, D), jnp.float32),
            pltpu.VMEM((L, D), jnp.float32),
            pltpu.VMEM((L, D), jnp.float32),
            pltpu.VMEM((L, D), jnp.float32),
            pltpu.VMEM((rpw, D), jnp.float32),
            pltpu.VMEM((rpw,), jnp.float32),
            pltpu.SemaphoreType.DMA,
            pltpu.SemaphoreType.DMA,
            pltpu.SemaphoreType.DMA,
            pltpu.SemaphoreType.DMA,
        ],
    )
    def run(ids_hbm, table_hbm, out_hbm, ids_v, buf0, buf1, buf2, buf3,
            out_v, inv_v, sem0, sem1, sem2, sem3):
        bufs = (buf0, buf1, buf2, buf3)
        sems = (sem0, sem1, sem2, sem3)
        nbuf = len(bufs)
        wid = lax.axis_index("s") * NUM_CORES + lax.axis_index("c")
        row0 = wid * rpw
        pltpu.sync_copy(ids_hbm.at[pl.ds(row0 * LP, rpw * LP)], ids_v)

        def issue(r, buf, sem):
            off = r * LP
            pltpu.async_copy(
                table_hbm.at[ids_v.at[pl.ds(off, 128)]],
                buf.at[pl.ds(0, 128)], sem)
            pltpu.async_copy(
                table_hbm.at[ids_v.at[pl.ds(off + 128, l_hi)]],
                buf.at[pl.ds(128, l_hi)], sem)

        def wait_buf(buf, sem):
            pltpu.make_async_copy(table_hbm.at[pl.ds(0, L)], buf, sem).wait()

        lane = lax.iota(jnp.int32, LANES)
        one = jnp.ones(LANES, jnp.float32)
        zero = jnp.zeros(LANES, jnp.float32)

        def count_group(g, carry):
            rowoff = lane * LP + g * (LANES * LP)

            def cbody(j, cnt):
                v = plsc.load_gather(ids_v, [rowoff + j])
                return cnt + jnp.where(v != 0, one, zero)
            cnt = lax.fori_loop(0, L, cbody, zero, unroll=8)
            inv_v[pl.ds(g * LANES, LANES)] = 1.0 / jnp.maximum(cnt, 1.0)
            return carry

        def compute(r, buf):
            def sum_body(j, accs):
                a0, a1 = accs
                return (a0 + buf[j, pl.ds(0, LANES)],
                        a1 + buf[j, pl.ds(LANES, LANES)])
            a0, a1 = lax.fori_loop(
                0, L, sum_body,
                (jnp.zeros(LANES, jnp.float32), jnp.zeros(LANES, jnp.float32)),
                unroll=8)
            inv = plsc.load_gather(inv_v, [jnp.full((LANES,), r, jnp.int32)])
            out_v[r, pl.ds(0, LANES)] = a0 * inv
            out_v[r, pl.ds(LANES, LANES)] = a1 * inv

        for k in range(nbuf):
            issue(k, bufs[k], sems[k])
        lax.fori_loop(0, rpw // LANES, count_group, 0)

        def outer(g, carry):
            r0 = g * nbuf
            for k in range(nbuf):
                wait_buf(bufs[k], sems[k])
                compute(r0 + k, bufs[k])

                @pl.when(r0 + k + nbuf < rpw)
                def _():
                    issue(r0 + k + nbuf, bufs[k], sems[k])
            return carry

        lax.fori_loop(0, rpw // nbuf, outer, 0)
        pltpu.sync_copy(out_v, out_hbm.at[pl.ds(row0, rpw)])

    return run


def kernel(token_ids, table):
    B, L = token_ids.shape
    V, D = table.shape
    lp = 256
    ids_packed = _make_ids_packer(B, L, lp)(token_ids.T.astype(jnp.int32))
    packed = _make_packer(V, D)(table.T)
    vp = packed.shape[0] // D
    run = _make_kernel(B, L, lp, vp, D)
    return run(ids_packed, packed.reshape(vp, D))


# speedup vs baseline: 8.8136x; 1.0530x over previous
import functools

import jax
import jax.numpy as jnp
from jax import lax
from jax.experimental import pallas as pl
from jax.experimental.pallas import tpu as pltpu
from jax.experimental.pallas import tpu_sc as plsc

NUM_CORES = 2
NUM_SUBCORES = 16
NUM_WORKERS = NUM_CORES * NUM_SUBCORES
LANES = 16


PACK_BR = 65536


def _make_packer(V, D):
    BR = PACK_BR
    nblk = (V + BR - 1) // BR
    CH = 2048
    S = CH // 4

    def body(in_ref, out_ref):
        for c in range(BR // CH):
            x = in_ref[:, pl.ds(c * CH, CH)]
            z = jnp.concatenate(
                [x[:, b * S:(b + 1) * S] for b in range(4)], axis=0)
            out_ref[pl.ds(c * CH * D, CH * D)] = (
                jnp.transpose(z).reshape(CH * D))

    return pl.pallas_call(
        body,
        grid=(nblk,),
        in_specs=[pl.BlockSpec((D, BR), lambda i: (0, i))],
        out_specs=pl.BlockSpec((BR * D,), lambda i: (i,)),
        out_shape=jax.ShapeDtypeStruct((nblk * BR * D,), jnp.float32),
    )


def _remap_ids(ids):
    h = ids >> 11
    q = ids & 2047
    b = q >> 9
    i = q & 511
    return (h << 11) + (i << 2) + b


def _make_ids_packer(B, L, LP):
    BB = 512

    def body(in_ref, out_ref, inv_ref):
        p = _remap_ids(in_ref[...])
        z = jnp.concatenate(
            [p, jnp.zeros((LP - L, BB), jnp.int32)], axis=0)
        out_ref[...] = jnp.transpose(z).reshape(BB * LP)
        cnt = jnp.sum((p != 0).astype(jnp.float32), axis=0)
        inv_ref[...] = 1.0 / jnp.maximum(cnt, 1.0)

    return pl.pallas_call(
        body,
        grid=(B // BB,),
        in_specs=[pl.BlockSpec((L, BB), lambda i: (0, i))],
        out_specs=[pl.BlockSpec((BB * LP,), lambda i: (i,)),
                   pl.BlockSpec((BB,), lambda i: (i,))],
        out_shape=[jax.ShapeDtypeStruct((B * LP,), jnp.int32),
                   jax.ShapeDtypeStruct((B,), jnp.float32)],
    )


def _make_kernel(B, L, LP, V, D):
    rpw = B // NUM_WORKERS
    assert B % NUM_WORKERS == 0
    assert D == 2 * LANES
    assert L % 8 == 0 and L > 128 and L <= 256
    l_hi = L - 128
    n_full = L // LANES
    l_tail = L - n_full * LANES

    mesh = plsc.VectorSubcoreMesh(core_axis_name="c", subcore_axis_name="s")

    @functools.partial(
        pl.kernel,
        out_type=jax.ShapeDtypeStruct((B, D), jnp.float32),
        mesh=mesh,
        compiler_params=pltpu.CompilerParams(
            needs_layout_passes=False, use_tc_tiling_on_sc=False),
        scratch_types=[
            pltpu.VMEM((rpw * LP,), jnp.int32),
            pltpu.VMEM((L, D), jnp.float32),
            pltpu.VMEM((L, D), jnp.float32),
            pltpu.VMEM((L, D), jnp.float32),
            pltpu.VMEM((L, D), jnp.float32),
            pltpu.VMEM((rpw, D), jnp.float32),
            pltpu.VMEM((rpw,), jnp.float32),
            pltpu.SemaphoreType.DMA,
            pltpu.SemaphoreType.DMA,
            pltpu.SemaphoreType.DMA,
            pltpu.SemaphoreType.DMA,
        ],
    )
    def run(ids_hbm, invd_hbm, table_hbm, out_hbm, ids_v, buf0, buf1, buf2,
            buf3, out_v, inv_v, sem0, sem1, sem2, sem3):
        bufs = (buf0, buf1, buf2, buf3)
        sems = (sem0, sem1, sem2, sem3)
        nbuf = len(bufs)
        wid = lax.axis_index("s") * NUM_CORES + lax.axis_index("c")
        row0 = wid * rpw
        pltpu.sync_copy(ids_hbm.at[pl.ds(row0 * LP, rpw * LP)], ids_v)
        pltpu.sync_copy(invd_hbm.at[pl.ds(row0, rpw)], inv_v)

        def issue(r, buf, sem):
            off = r * LP
            pltpu.async_copy(
                table_hbm.at[ids_v.at[pl.ds(off, 128)]],
                buf.at[pl.ds(0, 128)], sem)
            pltpu.async_copy(
                table_hbm.at[ids_v.at[pl.ds(off + 128, l_hi)]],
                buf.at[pl.ds(128, l_hi)], sem)

        def wait_buf(buf, sem):
            pltpu.make_async_copy(table_hbm.at[pl.ds(0, L)], buf, sem).wait()

        def compute(r, buf):
            def sum_body(j, accs):
                a0, a1 = accs
                return (a0 + buf[j, pl.ds(0, LANES)],
                        a1 + buf[j, pl.ds(LANES, LANES)])
            a0, a1 = lax.fori_loop(
                0, L, sum_body,
                (jnp.zeros(LANES, jnp.float32), jnp.zeros(LANES, jnp.float32)),
                unroll=8)
            inv = plsc.load_gather(inv_v, [jnp.full((LANES,), r, jnp.int32)])
            out_v[r, pl.ds(0, LANES)] = a0 * inv
            out_v[r, pl.ds(LANES, LANES)] = a1 * inv

        for k in range(nbuf):
            issue(k, bufs[k], sems[k])

        def outer(g, carry):
            r0 = g * nbuf
            for k in range(nbuf):
                wait_buf(bufs[k], sems[k])
                compute(r0 + k, bufs[k])

                @pl.when(r0 + k + nbuf < rpw)
                def _():
                    issue(r0 + k + nbuf, bufs[k], sems[k])
            return carry

        lax.fori_loop(0, rpw // nbuf, outer, 0)
        pltpu.sync_copy(out_v, out_hbm.at[pl.ds(row0, rpw)])

    return run


def kernel(token_ids, table):
    B, L = token_ids.shape
    V, D = table.shape
    lp = 256
    ids_packed, invd = _make_ids_packer(B, L, lp)(token_ids.T.astype(jnp.int32))
    packed = _make_packer(V, D)(table.T)
    vp = packed.shape[0] // D
    run = _make_kernel(B, L, lp, vp, D)
    return run(ids_packed, invd, packed.reshape(vp, D))


# speedup vs baseline: 9.3178x; 1.0572x over previous
import functools

import jax
import jax.numpy as jnp
from jax import lax
from jax.experimental import pallas as pl
from jax.experimental.pallas import tpu as pltpu
from jax.experimental.pallas import tpu_sc as plsc

NUM_CORES = 2
NUM_SUBCORES = 16
NUM_WORKERS = NUM_CORES * NUM_SUBCORES
LANES = 16


PACK_BR = 65536


def _make_packer(V, D):
    BR = PACK_BR
    nblk = (V + BR - 1) // BR
    CH = 2048
    S = CH // 4

    def body(in_ref, out_ref):
        for c in range(BR // CH):
            x = in_ref[:, pl.ds(c * CH, CH)]
            z = jnp.concatenate(
                [x[:, b * S:(b + 1) * S] for b in range(4)], axis=0)
            out_ref[pl.ds(c * CH * D, CH * D)] = (
                jnp.transpose(z).reshape(CH * D))

    return pl.pallas_call(
        body,
        grid=(nblk,),
        in_specs=[pl.BlockSpec((D, BR), lambda i: (0, i))],
        out_specs=pl.BlockSpec((BR * D,), lambda i: (i,)),
        out_shape=jax.ShapeDtypeStruct((nblk * BR * D,), jnp.float32),
    )


def _remap_ids(ids):
    h = ids >> 11
    q = ids & 2047
    b = q >> 9
    i = q & 511
    return (h << 11) + (i << 2) + b


def _make_ids_packer(B, L, LP):
    BB = 512

    def body(in_ref, out_ref, inv_ref):
        p = _remap_ids(in_ref[...])
        z = jnp.concatenate(
            [p, jnp.zeros((LP - L, BB), jnp.int32)], axis=0)
        out_ref[...] = jnp.transpose(z).reshape(BB * LP)
        cnt = jnp.sum((p != 0).astype(jnp.float32), axis=0)
        inv_ref[...] = 1.0 / jnp.maximum(cnt, 1.0)

    return pl.pallas_call(
        body,
        grid=(B // BB,),
        in_specs=[pl.BlockSpec((L, BB), lambda i: (0, i))],
        out_specs=[pl.BlockSpec((BB * LP,), lambda i: (i,)),
                   pl.BlockSpec((BB,), lambda i: (i,))],
        out_shape=[jax.ShapeDtypeStruct((B * LP,), jnp.int32),
                   jax.ShapeDtypeStruct((B,), jnp.float32)],
    )


def _make_kernel(B, L, LP, V, D):
    rpw = B // NUM_WORKERS
    assert B % NUM_WORKERS == 0
    assert D == 2 * LANES
    assert L % 8 == 0 and L > 128 and L <= 256
    l_hi = L - 128
    n_full = L // LANES
    l_tail = L - n_full * LANES

    mesh = plsc.VectorSubcoreMesh(core_axis_name="c", subcore_axis_name="s")

    @functools.partial(
        pl.kernel,
        out_type=jax.ShapeDtypeStruct((B, D), jnp.float32),
        mesh=mesh,
        compiler_params=pltpu.CompilerParams(
            needs_layout_passes=False, use_tc_tiling_on_sc=False),
        scratch_types=[
            pltpu.VMEM((rpw * LP,), jnp.int32),
            pltpu.VMEM((L, D), jnp.float32),
            pltpu.VMEM((L, D), jnp.float32),
            pltpu.VMEM((L, D), jnp.float32),
            pltpu.VMEM((L, D), jnp.float32),
            pltpu.VMEM((L, D), jnp.float32),
            pltpu.VMEM((L, D), jnp.float32),
            pltpu.VMEM((L, D), jnp.float32),
            pltpu.VMEM((L, D), jnp.float32),
            pltpu.VMEM((rpw, D), jnp.float32),
            pltpu.VMEM((rpw,), jnp.float32),
            pltpu.SemaphoreType.DMA,
            pltpu.SemaphoreType.DMA,
            pltpu.SemaphoreType.DMA,
            pltpu.SemaphoreType.DMA,
            pltpu.SemaphoreType.DMA,
            pltpu.SemaphoreType.DMA,
            pltpu.SemaphoreType.DMA,
            pltpu.SemaphoreType.DMA,
        ],
    )
    def run(ids_hbm, invd_hbm, table_hbm, out_hbm, ids_v, buf0, buf1, buf2,
            buf3, buf4, buf5, buf6, buf7, out_v, inv_v,
            sem0, sem1, sem2, sem3, sem4, sem5, sem6, sem7):
        bufs = (buf0, buf1, buf2, buf3, buf4, buf5, buf6, buf7)
        sems = (sem0, sem1, sem2, sem3, sem4, sem5, sem6, sem7)
        nbuf = len(bufs)
        wid = lax.axis_index("s") * NUM_CORES + lax.axis_index("c")
        row0 = wid * rpw
        pltpu.sync_copy(ids_hbm.at[pl.ds(row0 * LP, rpw * LP)], ids_v)
        pltpu.sync_copy(invd_hbm.at[pl.ds(row0, rpw)], inv_v)

        def issue(r, buf, sem):
            off = r * LP
            pltpu.async_copy(
                table_hbm.at[ids_v.at[pl.ds(off, 128)]],
                buf.at[pl.ds(0, 128)], sem)
            pltpu.async_copy(
                table_hbm.at[ids_v.at[pl.ds(off + 128, l_hi)]],
                buf.at[pl.ds(128, l_hi)], sem)

        def wait_buf(buf, sem):
            pltpu.make_async_copy(table_hbm.at[pl.ds(0, L)], buf, sem).wait()

        def compute(r, buf):
            def sum_body(j, accs):
                a0, a1 = accs
                return (a0 + buf[j, pl.ds(0, LANES)],
                        a1 + buf[j, pl.ds(LANES, LANES)])
            a0, a1 = lax.fori_loop(
                0, L, sum_body,
                (jnp.zeros(LANES, jnp.float32), jnp.zeros(LANES, jnp.float32)),
                unroll=8)
            inv = plsc.load_gather(inv_v, [jnp.full((LANES,), r, jnp.int32)])
            out_v[r, pl.ds(0, LANES)] = a0 * inv
            out_v[r, pl.ds(LANES, LANES)] = a1 * inv

        for k in range(nbuf):
            issue(k, bufs[k], sems[k])

        def outer(g, carry):
            r0 = g * nbuf
            for k in range(nbuf):
                wait_buf(bufs[k], sems[k])
                compute(r0 + k, bufs[k])

                @pl.when(r0 + k + nbuf < rpw)
                def _():
                    issue(r0 + k + nbuf, bufs[k], sems[k])
            return carry

        lax.fori_loop(0, rpw // nbuf, outer, 0)
        pltpu.sync_copy(out_v, out_hbm.at[pl.ds(row0, rpw)])

    return run


def kernel(token_ids, table):
    B, L = token_ids.shape
    V, D = table.shape
    lp = 256
    ids_packed, invd = _make_ids_packer(B, L, lp)(token_ids.T.astype(jnp.int32))
    packed = _make_packer(V, D)(table.T)
    vp = packed.shape[0] // D
    run = _make_kernel(B, L, lp, vp, D)
    return run(ids_packed, invd, packed.reshape(vp, D))
